# baseline (speedup 1.0000x reference)
import jax
import jax.numpy as jnp
from jax.experimental import pallas as pl
from jax.experimental.pallas import tpu as pltpu


def _conv3x3_flat(y, w_ref, n_out):
    y = y.astype(jnp.bfloat16)
    H, W, C = y.shape
    zrow = jnp.zeros((1, W, C), y.dtype)
    yp = jnp.concatenate([zrow, y, zrow], axis=0)
    zcol = jnp.zeros((H + 2, 1, C), y.dtype)
    yl = jnp.concatenate([yp[:, 1:, :], zcol], axis=1)
    yr = jnp.concatenate([zcol, yp[:, : W - 1, :]], axis=1)
    srcs = [yr.reshape((H + 2) * W, C),
            yp.reshape((H + 2) * W, C),
            yl.reshape((H + 2) * W, C)]
    acc = jnp.zeros((H * W, n_out), jnp.float32)
    for i in range(3):
        for j in range(3):
            sl = srcs[j][i * W: i * W + H * W, :]
            acc = acc + jnp.dot(sl, w_ref[i, j],
                                preferred_element_type=jnp.float32)
    return acc


def _apply_affine(x, a_ref):
    y = x * a_ref[0] + a_ref[1]
    return jnp.where(y >= 0.0, y, a_ref[2] * y)


def _convt_body(xd_ref, w_ref, o_ref, s_ref, sq_ref):
    xd = xd_ref[0]
    Hp, Wp, C = xd.shape
    y = xd[1:Hp - 1, 1:Wp - 1, :]
    acc = _conv3x3_flat(y, w_ref, C)
    o_ref[0] = acc.reshape(Hp - 2, Wp - 2, C)
    s_ref[0] = jnp.sum(acc, axis=0, keepdims=True)
    sq_ref[0] = jnp.sum(acc * acc, axis=0, keepdims=True)


def _conv0_body(sk_ref, up_ref, a_ref, w_ref, o_ref, s_ref, sq_ref):
    u = _apply_affine(up_ref[0], a_ref)
    y = jnp.concatenate([sk_ref[0], u], axis=-1)
    n_out = o_ref.shape[-1]
    acc = _conv3x3_flat(y, w_ref, n_out)
    H, W, _ = y.shape
    o_ref[0] = acc.reshape(H, W, n_out)
    s_ref[0] = jnp.sum(acc, axis=0, keepdims=True)
    sq_ref[0] = jnp.sum(acc * acc, axis=0, keepdims=True)


def _conv_body(x_ref, a_ref, w_ref, o_ref, s_ref, sq_ref):
    y = _apply_affine(x_ref[0], a_ref)
    n_out = o_ref.shape[-1]
    acc = _conv3x3_flat(y, w_ref, n_out)
    H, W, _ = y.shape
    o_ref[0] = acc.reshape(H, W, n_out)
    s_ref[0] = jnp.sum(acc, axis=0, keepdims=True)
    sq_ref[0] = jnp.sum(acc * acc, axis=0, keepdims=True)


def _resid_body(y_ref, ay_ref, cb_ref, acb_ref, w_ref, b_ref, o_ref):
    y = _apply_affine(y_ref[0], ay_ref)
    cb = _apply_affine(cb_ref[0], acb_ref)
    H, W, C = y.shape
    n = w_ref.shape[1]
    out = jnp.dot(y.astype(jnp.bfloat16).reshape(H * W, C), w_ref[...],
                  preferred_element_type=jnp.float32)
    o_ref[0] = out.reshape(H, W, n) + b_ref[0] + cb


_PAR = pltpu.CompilerParams(dimension_semantics=("parallel",))


def _conv_stage(body, tensors, specs, out_shapes, out_specs, B):
    return pl.pallas_call(
        body,
        out_shape=out_shapes,
        grid=(B,),
        in_specs=specs,
        out_specs=out_specs,
        compiler_params=_PAR,
    )(*tensors)


def _stats_affine(s, sq, gamma, beta, alpha, count, eps=1e-5):
    tot = jnp.sum(s, axis=(0, 1))
    tot_sq = jnp.sum(sq, axis=(0, 1))
    mean = tot / count
    var = jnp.maximum(tot_sq / count - mean * mean, 0.0)
    scale = gamma / jnp.sqrt(var + eps)
    shift = beta - mean * scale
    a = jnp.broadcast_to(jnp.asarray(alpha, jnp.float32).reshape(()), scale.shape)
    return jnp.stack([scale, shift, a], axis=0)


def kernel(convT_w, convT_b, bn0_g, bn0_b, prelu0_a, conv0_w, conv0_b,
           bn1_g, bn1_b, prelu1_a, rb_c1_w, rb_c1_b, rb_bn1_g, rb_bn1_b,
           rb_prelu1_a, rb_c2_w, rb_c2_b, rb_bn2_g, rb_bn2_b, rb_prelu2_a,
           rb_c11_w, rb_c11_b, x, skip):
    xh = jnp.transpose(x, (0, 2, 3, 1)).astype(jnp.float32)
    sk = jnp.transpose(skip, (0, 2, 3, 1)).astype(jnp.float32)
    B, H, W, c = xh.shape
    Hs, Ws = sk.shape[1], sk.shape[2]
    n = conv0_w.shape[0]
    count = B * Hs * Ws

    op_h = Hs - ((H - 1) * 2 - 2 + 3)
    op_w = Ws - ((W - 1) * 2 - 2 + 3)
    hd, wd = 2 * H - 1, 2 * W - 1
    xd = jnp.zeros((B, hd, wd, c), jnp.float32).at[:, ::2, ::2, :].set(xh)
    xd = jnp.pad(xd, ((0, 0), (1, 1 + op_h), (1, 1 + op_w), (0, 0)))

    wt = jnp.transpose(jnp.flip(convT_w, (2, 3)), (2, 3, 0, 1)).astype(
        jnp.bfloat16)
    w0 = jnp.transpose(conv0_w, (2, 3, 1, 0)).astype(jnp.bfloat16)
    w1 = jnp.transpose(rb_c1_w, (2, 3, 1, 0)).astype(jnp.bfloat16)
    w2 = jnp.transpose(rb_c2_w, (2, 3, 1, 0)).astype(jnp.bfloat16)
    w11 = rb_c11_w[:, :, 0, 0].T.astype(jnp.bfloat16)
    b11 = rb_c11_b.reshape(1, -1).astype(jnp.float32)

    full = lambda *dims: pl.BlockSpec(dims, lambda b: (0,) * len(dims))
    perb = lambda *dims: pl.BlockSpec((1,) + dims,
                                      lambda b: (b,) + (0,) * len(dims))
    stat_shapes = lambda ch: (jax.ShapeDtypeStruct((B, 1, ch), jnp.float32),
                              jax.ShapeDtypeStruct((B, 1, ch), jnp.float32))

    def sspec(ch):
        return pl.BlockSpec((1, 1, ch), lambda b: (b, 0, 0))

    up, s, sq = _conv_stage(
        _convt_body, (xd, wt),
        [perb(Hs + 2, Ws + 2, c), full(3, 3, c, c)],
        (jax.ShapeDtypeStruct((B, Hs, Ws, c), jnp.float32),) + stat_shapes(c),
        (perb(Hs, Ws, c), sspec(c), sspec(c)), B)
    aff0 = _stats_affine(s, sq, bn0_g, bn0_b, prelu0_a, count)

    y_raw, s, sq = _conv_stage(
        _conv0_body, (sk, up, aff0, w0),
        [perb(Hs, Ws, c), perb(Hs, Ws, c), full(3, c), full(3, 3, 2 * c, n)],
        (jax.ShapeDtypeStruct((B, Hs, Ws, n), jnp.float32),) + stat_shapes(n),
        (perb(Hs, Ws, n), sspec(n), sspec(n)), B)
    aff_y = _stats_affine(s, sq, bn1_g, bn1_b, prelu1_a, count)

    cb1, s, sq = _conv_stage(
        _conv_body, (y_raw, aff_y, w1),
        [perb(Hs, Ws, n), full(3, n), full(3, 3, n, n)],
        (jax.ShapeDtypeStruct((B, Hs, Ws, n), jnp.float32),) + stat_shapes(n),
        (perb(Hs, Ws, n), sspec(n), sspec(n)), B)
    aff1 = _stats_affine(s, sq, rb_bn1_g, rb_bn1_b, rb_prelu1_a, count)

    cb2, s, sq = _conv_stage(
        _conv_body, (cb1, aff1, w2),
        [perb(Hs, Ws, n), full(3, n), full(3, 3, n, n)],
        (jax.ShapeDtypeStruct((B, Hs, Ws, n), jnp.float32),) + stat_shapes(n),
        (perb(Hs, Ws, n), sspec(n), sspec(n)), B)
    aff2 = _stats_affine(s, sq, rb_bn2_g, rb_bn2_b, rb_prelu2_a, count)

    out = _conv_stage(
        _resid_body, (y_raw, aff_y, cb2, aff2, w11, b11),
        [perb(Hs, Ws, n), full(3, n), perb(Hs, Ws, n), full(3, n),
         full(n, n), full(1, n)],
        jax.ShapeDtypeStruct((B, Hs, Ws, n), jnp.float32),
        perb(Hs, Ws, n), B)
    return jnp.transpose(out, (0, 3, 1, 2))


# speedup vs baseline: 1.0317x; 1.0317x over previous
import jax
import jax.numpy as jnp
from jax.experimental import pallas as pl
from jax.experimental.pallas import tpu as pltpu


def _conv3x3_flat(y, w_ref, n_out):
    y = y.astype(jnp.bfloat16)
    H, W, C = y.shape
    zrow = jnp.zeros((1, W, C), y.dtype)
    yp = jnp.concatenate([zrow, y, zrow], axis=0)
    zcol = jnp.zeros((H + 2, 1, C), y.dtype)
    yl = jnp.concatenate([yp[:, 1:, :], zcol], axis=1)
    yr = jnp.concatenate([zcol, yp[:, : W - 1, :]], axis=1)
    srcs = [yr.reshape((H + 2) * W, C),
            yp.reshape((H + 2) * W, C),
            yl.reshape((H + 2) * W, C)]
    cols = jnp.concatenate(
        [srcs[j][i * W: i * W + H * W, :] for i in range(3) for j in range(3)],
        axis=-1)
    return jnp.dot(cols, w_ref[...], preferred_element_type=jnp.float32)


def _apply_affine(x, a_ref):
    y = x * a_ref[0] + a_ref[1]
    return jnp.where(y >= 0.0, y, a_ref[2] * y)


def _convt_body(xd_ref, w_ref, o_ref, s_ref, sq_ref):
    xd = xd_ref[0]
    Hp, Wp, C = xd.shape
    y = xd[1:Hp - 1, 1:Wp - 1, :]
    acc = _conv3x3_flat(y, w_ref, C)
    o_ref[0] = acc.reshape(Hp - 2, Wp - 2, C)
    s_ref[0] = jnp.sum(acc, axis=0, keepdims=True)
    sq_ref[0] = jnp.sum(acc * acc, axis=0, keepdims=True)


def _conv0_body(sk_ref, up_ref, a_ref, w_ref, o_ref, s_ref, sq_ref):
    u = _apply_affine(up_ref[0], a_ref)
    y = jnp.concatenate([sk_ref[0], u], axis=-1)
    n_out = o_ref.shape[-1]
    acc = _conv3x3_flat(y, w_ref, n_out)
    H, W, _ = y.shape
    o_ref[0] = acc.reshape(H, W, n_out)
    s_ref[0] = jnp.sum(acc, axis=0, keepdims=True)
    sq_ref[0] = jnp.sum(acc * acc, axis=0, keepdims=True)


def _conv_body(x_ref, a_ref, w_ref, o_ref, s_ref, sq_ref):
    y = _apply_affine(x_ref[0], a_ref)
    n_out = o_ref.shape[-1]
    acc = _conv3x3_flat(y, w_ref, n_out)
    H, W, _ = y.shape
    o_ref[0] = acc.reshape(H, W, n_out)
    s_ref[0] = jnp.sum(acc, axis=0, keepdims=True)
    sq_ref[0] = jnp.sum(acc * acc, axis=0, keepdims=True)


def _resid_body(y_ref, ay_ref, cb_ref, acb_ref, w_ref, b_ref, o_ref):
    y = _apply_affine(y_ref[0], ay_ref)
    cb = _apply_affine(cb_ref[0], acb_ref)
    H, W, C = y.shape
    n = w_ref.shape[1]
    out = jnp.dot(y.astype(jnp.bfloat16).reshape(H * W, C), w_ref[...],
                  preferred_element_type=jnp.float32)
    o_ref[0] = out.reshape(H, W, n) + b_ref[0] + cb


_PAR = pltpu.CompilerParams(dimension_semantics=("parallel",))


def _conv_stage(body, tensors, specs, out_shapes, out_specs, B):
    return pl.pallas_call(
        body,
        out_shape=out_shapes,
        grid=(B,),
        in_specs=specs,
        out_specs=out_specs,
        compiler_params=_PAR,
    )(*tensors)


def _stats_affine(s, sq, gamma, beta, alpha, count, eps=1e-5):
    tot = jnp.sum(s, axis=(0, 1))
    tot_sq = jnp.sum(sq, axis=(0, 1))
    mean = tot / count
    var = jnp.maximum(tot_sq / count - mean * mean, 0.0)
    scale = gamma / jnp.sqrt(var + eps)
    shift = beta - mean * scale
    a = jnp.broadcast_to(jnp.asarray(alpha, jnp.float32).reshape(()), scale.shape)
    return jnp.stack([scale, shift, a], axis=0)


def kernel(convT_w, convT_b, bn0_g, bn0_b, prelu0_a, conv0_w, conv0_b,
           bn1_g, bn1_b, prelu1_a, rb_c1_w, rb_c1_b, rb_bn1_g, rb_bn1_b,
           rb_prelu1_a, rb_c2_w, rb_c2_b, rb_bn2_g, rb_bn2_b, rb_prelu2_a,
           rb_c11_w, rb_c11_b, x, skip):
    xh = jnp.transpose(x, (0, 2, 3, 1)).astype(jnp.float32)
    sk = jnp.transpose(skip, (0, 2, 3, 1)).astype(jnp.float32)
    B, H, W, c = xh.shape
    Hs, Ws = sk.shape[1], sk.shape[2]
    n = conv0_w.shape[0]
    count = B * Hs * Ws

    op_h = Hs - ((H - 1) * 2 - 2 + 3)
    op_w = Ws - ((W - 1) * 2 - 2 + 3)
    hd, wd = 2 * H - 1, 2 * W - 1
    xd = jnp.zeros((B, hd, wd, c), jnp.float32).at[:, ::2, ::2, :].set(xh)
    xd = jnp.pad(xd, ((0, 0), (1, 1 + op_h), (1, 1 + op_w), (0, 0)))

    wt = jnp.transpose(jnp.flip(convT_w, (2, 3)), (2, 3, 0, 1)).astype(
        jnp.bfloat16).reshape(9 * c, c)
    w0 = jnp.transpose(conv0_w, (2, 3, 1, 0)).astype(jnp.bfloat16).reshape(
        18 * c, n)
    w1 = jnp.transpose(rb_c1_w, (2, 3, 1, 0)).astype(jnp.bfloat16).reshape(9 * n, n)
    w2 = jnp.transpose(rb_c2_w, (2, 3, 1, 0)).astype(jnp.bfloat16).reshape(9 * n, n)
    w11 = rb_c11_w[:, :, 0, 0].T.astype(jnp.bfloat16)
    b11 = rb_c11_b.reshape(1, -1).astype(jnp.float32)

    full = lambda *dims: pl.BlockSpec(dims, lambda b: (0,) * len(dims))
    perb = lambda *dims: pl.BlockSpec((1,) + dims,
                                      lambda b: (b,) + (0,) * len(dims))
    stat_shapes = lambda ch: (jax.ShapeDtypeStruct((B, 1, ch), jnp.float32),
                              jax.ShapeDtypeStruct((B, 1, ch), jnp.float32))

    def sspec(ch):
        return pl.BlockSpec((1, 1, ch), lambda b: (b, 0, 0))

    up, s, sq = _conv_stage(
        _convt_body, (xd, wt),
        [perb(Hs + 2, Ws + 2, c), full(9 * c, c)],
        (jax.ShapeDtypeStruct((B, Hs, Ws, c), jnp.float32),) + stat_shapes(c),
        (perb(Hs, Ws, c), sspec(c), sspec(c)), B)
    aff0 = _stats_affine(s, sq, bn0_g, bn0_b, prelu0_a, count)

    y_raw, s, sq = _conv_stage(
        _conv0_body, (sk, up, aff0, w0),
        [perb(Hs, Ws, c), perb(Hs, Ws, c), full(3, c), full(18 * c, n)],
        (jax.ShapeDtypeStruct((B, Hs, Ws, n), jnp.float32),) + stat_shapes(n),
        (perb(Hs, Ws, n), sspec(n), sspec(n)), B)
    aff_y = _stats_affine(s, sq, bn1_g, bn1_b, prelu1_a, count)

    cb1, s, sq = _conv_stage(
        _conv_body, (y_raw, aff_y, w1),
        [perb(Hs, Ws, n), full(3, n), full(9 * n, n)],
        (jax.ShapeDtypeStruct((B, Hs, Ws, n), jnp.float32),) + stat_shapes(n),
        (perb(Hs, Ws, n), sspec(n), sspec(n)), B)
    aff1 = _stats_affine(s, sq, rb_bn1_g, rb_bn1_b, rb_prelu1_a, count)

    cb2, s, sq = _conv_stage(
        _conv_body, (cb1, aff1, w2),
        [perb(Hs, Ws, n), full(3, n), full(9 * n, n)],
        (jax.ShapeDtypeStruct((B, Hs, Ws, n), jnp.float32),) + stat_shapes(n),
        (perb(Hs, Ws, n), sspec(n), sspec(n)), B)
    aff2 = _stats_affine(s, sq, rb_bn2_g, rb_bn2_b, rb_prelu2_a, count)

    out = _conv_stage(
        _resid_body, (y_raw, aff_y, cb2, aff2, w11, b11),
        [perb(Hs, Ws, n), full(3, n), perb(Hs, Ws, n), full(3, n),
         full(n, n), full(1, n)],
        jax.ShapeDtypeStruct((B, Hs, Ws, n), jnp.float32),
        perb(Hs, Ws, n), B)
    return jnp.transpose(out, (0, 3, 1, 2))
